# Initial kernel scaffold; baseline (speedup 1.0000x reference)
import jax
import jax.numpy as jnp
from jax.experimental import pallas as pl


def kernel(pe, positions):
    raise NotImplementedError("write your pallas kernel here")


# speedup vs baseline: 6.3048x; 6.3048x over previous
import functools

import jax
import jax.numpy as jnp
from jax import lax
from jax.experimental import pallas as pl
from jax.experimental.pallas import tpu as pltpu
from jax.experimental.pallas import tpu_sc as plsc

_CH = 128
_B = 4096 * 200
_NC = 2
_NS = 16
_NW = _NC * _NS
_BPW = _B // _NW
_CHUNK = 128
_NSTEP = _BPW // _CHUNK


@functools.partial(
    pl.kernel,
    mesh=plsc.VectorSubcoreMesh(core_axis_name="c", subcore_axis_name="s"),
    out_type=jax.ShapeDtypeStruct((_B, _CH), jnp.float32),
    scratch_types=[
        pltpu.VMEM((_NSTEP, _CHUNK), jnp.int32),
        pltpu.VMEM((_CHUNK, _CH), jnp.float32),
        pltpu.SemaphoreType.DMA,
    ],
)
def _pe_gather(pe_hbm, pos_hbm, out_hbm, idx_v, rows_v, sem):
    wid = lax.axis_index("s") * _NC + lax.axis_index("c")
    base = wid * _BPW
    pltpu.sync_copy(pos_hbm.at[wid], idx_v)

    def step(j, carry):
        pltpu.async_copy(pe_hbm.at[idx_v.at[j]], rows_v, sem).wait()
        pltpu.sync_copy(rows_v, out_hbm.at[pl.ds(base + j * _CHUNK, _CHUNK)])
        return carry

    lax.fori_loop(0, _NSTEP, step, 0)


def kernel(pe, positions):
    pos = positions.reshape(_NW, _NSTEP, _CHUNK)
    out = _pe_gather(pe, pos)
    return out.reshape(*positions.shape, _CH)


# speedup vs baseline: 9.1741x; 1.4551x over previous
import functools

import jax
import jax.numpy as jnp
from jax import lax
from jax.experimental import pallas as pl
from jax.experimental.pallas import tpu as pltpu
from jax.experimental.pallas import tpu_sc as plsc

_CH = 128
_B = 4096 * 200
_NC = 2
_NS = 16
_NW = _NC * _NS
_BPW = _B // _NW
_CHUNK = 128
_NSTEP = _BPW // _CHUNK
_K = 5
_NG = _NSTEP // _K


@functools.partial(
    pl.kernel,
    mesh=plsc.VectorSubcoreMesh(core_axis_name="c", subcore_axis_name="s"),
    out_type=jax.ShapeDtypeStruct((_B, _CH), jnp.float32),
    scratch_types=[
        pltpu.VMEM((_NSTEP, _CHUNK), jnp.int32),
        pltpu.VMEM((_K, _CHUNK, _CH), jnp.float32),
        pltpu.SemaphoreType.DMA,
        pltpu.SemaphoreType.DMA,
    ],
)
def _pe_gather(pe_hbm, pos_hbm, out_hbm, idx_v, rows_v, gsem, wsem):
    wid = lax.axis_index("s") * _NC + lax.axis_index("c")
    base = wid * _BPW
    pltpu.sync_copy(pos_hbm.at[wid], idx_v)

    def group(g, carry):
        j0 = g * _K
        gc = [
            pltpu.async_copy(pe_hbm.at[idx_v.at[j0 + b]], rows_v.at[b], gsem)
            for b in range(_K)
        ]
        wc = []
        for b in range(_K):
            gc[b].wait()
            wc.append(
                pltpu.async_copy(
                    rows_v.at[b],
                    out_hbm.at[pl.ds(base + (j0 + b) * _CHUNK, _CHUNK)],
                    wsem,
                )
            )
        for b in range(_K):
            wc[b].wait()
        return carry

    lax.fori_loop(0, _NG, group, 0)


def kernel(pe, positions):
    pos = positions.reshape(_NW, _NSTEP, _CHUNK)
    out = _pe_gather(pe, pos)
    return out.reshape(*positions.shape, _CH)
